# baseline (speedup 1.0000x reference)
import functools

import jax
import jax.numpy as jnp
from jax import lax
from jax.experimental import pallas as pl
from jax.experimental.pallas import tpu as pltpu
from jax.experimental.pallas import tpu_sc as plsc

N = 1000000
D = 16
C = 4
H = 512
W = 512
M = H * W

NC = 2
NS = 16
NW = NC * NS
BPW = M // NW
CH = 2048
NCHUNK = BPW // CH


def _sc_gather(ids, table):
    mesh = plsc.VectorSubcoreMesh(
        core_axis_name="c", subcore_axis_name="s", num_cores=NC, num_subcores=NS
    )

    @functools.partial(
        pl.kernel,
        out_type=jax.ShapeDtypeStruct((M, D), jnp.float32),
        mesh=mesh,
        scratch_types=[
            pltpu.VMEM((CH,), jnp.int32),
            pltpu.VMEM((CH, D), jnp.float32),
            pltpu.SemaphoreType.DMA,
        ],
        compiler_params=pltpu.CompilerParams(use_tc_tiling_on_sc=False),
    )
    def k(ids_hbm, table_hbm, out_hbm, idx_v, rows_v, sem):
        wid = lax.axis_index("s") * NC + lax.axis_index("c")
        base = wid * BPW
        for c in range(NCHUNK):
            off = base + c * CH
            pltpu.sync_copy(ids_hbm.at[pl.ds(off, CH)], idx_v)
            pltpu.async_copy(table_hbm.at[idx_v], rows_v, sem).wait()
            pltpu.sync_copy(rows_v, out_hbm.at[pl.ds(off, CH)])

    return k(ids, table)


BLK = 2048


def _tc_head_body(raw_ref, w_ref, hw_ref, b_ref, emb_ref, log_ref):
    emb = raw_ref[...] * w_ref[...]
    emb_ref[...] = emb
    log_ref[...] = (
        jnp.dot(emb, hw_ref[...], preferred_element_type=jnp.float32,
                precision=lax.Precision.HIGHEST)
        + b_ref[...]
    )


def _tc_head(raw, w, head_w, head_b):
    grid = M // BLK
    return pl.pallas_call(
        _tc_head_body,
        grid=(grid,),
        in_specs=[
            pl.BlockSpec((BLK, D), lambda i: (i, 0)),
            pl.BlockSpec((BLK, 1), lambda i: (i, 0)),
            pl.BlockSpec((D, C), lambda i: (0, 0)),
            pl.BlockSpec((1, C), lambda i: (0, 0)),
        ],
        out_specs=[
            pl.BlockSpec((BLK, D), lambda i: (i, 0)),
            pl.BlockSpec((BLK, C), lambda i: (i, 0)),
        ],
        out_shape=[
            jax.ShapeDtypeStruct((M, D), jnp.float32),
            jax.ShapeDtypeStruct((M, C), jnp.float32),
        ],
    )(raw, w, head_w, head_b)


def kernel(weights, gaussian_ids, semantic_features, head_w, head_b):
    ids = gaussian_ids.astype(jnp.int32)
    raw = _sc_gather(ids, semantic_features)
    emb, log = _tc_head(
        raw,
        weights.reshape(M, 1),
        head_w.T,
        head_b.reshape(1, C),
    )
    return emb.reshape(H, W, D), log.reshape(H, W, C)
